# baseline (speedup 1.0000x reference)
import functools

import jax
import jax.numpy as jnp
import numpy as np
from jax import lax
from jax.experimental import pallas as pl
from jax.experimental.pallas import tpu as pltpu, tpu_sc as plsc

N = 1048576
K_RANK = int(N * 0.3)
TEMP = 0.05
NC = 2
NS = 16
NW = NC * NS
E = N // NW
L = 16
NB1 = 2048
NB2 = 2048
NB3 = 1024
MIN_I32 = np.int32(-2147483648)

_mesh = plsc.VectorSubcoreMesh(core_axis_name="c", subcore_axis_name="s")


def _iota():
    return lax.broadcasted_iota(jnp.int32, (L,), 0)


def _splat(x, dtype=jnp.int32):
    return jnp.broadcast_to(jnp.asarray(x, dtype), (L,))


def _lane_splat(v, i):
    return jnp.broadcast_to(jnp.sum(jnp.where(_iota() == i, v, 0)), (L,))


def _sortable(x):
    u = plsc.bitcast(x, jnp.int32)
    m = lax.shift_right_arithmetic(u, _splat(31))
    return u ^ (m | MIN_I32)


def _srl(x, n):
    return lax.shift_right_logical(x, _splat(n))


def _hstride(nb):
    return nb + 1


def _zero_hist(hist_ref, nb):
    zeros = jnp.zeros((L,), jnp.int32)
    nwords = L * _hstride(nb)
    nfull = nwords // (L * L)

    def body(c, _):
        for u in range(L):
            hist_ref[pl.ds(c * (L * L) + u * L, L)] = zeros
        return 0

    lax.fori_loop(0, nfull, body, 0)
    for r in range(nfull * L * L, nwords, L):
        hist_ref[pl.ds(r, L)] = zeros


HIST_UNROLL = 16


def _histogram(data_ref, hist_ref, nb, shift, prefix_shift=None, prefix=None):
    lane_base = _iota() * _hstride(nb)
    ones = jnp.ones((L,), jnp.int32)
    step = L * HIST_UNROLL

    def body(i, _):
        base = i * step
        xs = [data_ref[pl.ds(base + u * L, L)] for u in range(HIST_UNROLL)]
        ss = [_sortable(x) for x in xs]
        bs = [lane_base + (_srl(s, shift) & (nb - 1)) for s in ss]
        if prefix is None:
            for b in bs:
                plsc.addupdate_scatter(hist_ref, [b], ones)
        else:
            ms = [_srl(s, prefix_shift) == prefix for s in ss]
            for b, m in zip(bs, ms):
                plsc.addupdate_scatter(hist_ref, [b], ones, mask=m)
        return 0

    lax.fori_loop(0, E // step, body, 0)


def _stride_reduce(hist_ref, red_ref, nb):
    hs = _hstride(nb)

    def body(c, _):
        acc = hist_ref[pl.ds(c * L, L)]
        for l in range(1, L):
            acc = acc + hist_ref[pl.ds(c * L + l * hs, L)]
        red_ref[pl.ds(c * L, L)] = acc
        return 0

    lax.fori_loop(0, nb // L, body, 0)


def _grid_reduce(grid_ref, red_ref, nb):
    def body(c, _):
        acc = grid_ref[0, pl.ds(c * L, L)]
        for l in range(1, L):
            acc = acc + grid_ref[l, pl.ds(c * L, L)]
        red_ref[pl.ds(c * L, L)] = acc
        return 0

    lax.fori_loop(0, nb // L, body, 0)


def _publish_hist(sid, cid, red_ref, shared_ref, stage_ref, out_hbm, nb,
                  sem):
    sl = nb // NS
    pltpu.sync_copy(red_ref, shared_ref.at[sid])
    plsc.subcore_barrier()
    hs = [
        pltpu.async_copy(
            shared_ref.at[l, pl.ds(sid * sl, sl)],
            stage_ref.at[l, pl.ds(0, sl)],
            sem,
        )
        for l in range(NS)
    ]
    for h in hs:
        h.wait()

    def body(c, _):
        acc = stage_ref[0, pl.ds(c * L, L)]
        for l in range(1, NS):
            acc = acc + stage_ref[l, pl.ds(c * L, L)]
        red_ref[pl.ds(c * L, L)] = acc
        return 0

    lax.fori_loop(0, sl // L, body, 0)
    pltpu.sync_copy(
        red_ref.at[pl.ds(0, sl)], out_hbm.at[cid, pl.ds(sid * sl, sl)]
    )


SUM_UNROLL = 8


def _sum_cores(hist_hbm, pair_ref, g_ref, nb):
    pltpu.sync_copy(hist_hbm, pair_ref)
    step = L * SUM_UNROLL

    def body(c, _):
        dss = [pl.ds(c * step + u * L, L) for u in range(SUM_UNROLL)]
        a = [pair_ref[0, ds] for ds in dss]
        b = [pair_ref[1, ds] for ds in dss]
        for u, ds in enumerate(dss):
            g_ref[ds] = a[u] + b[u]
        return 0

    lax.fori_loop(0, nb // step, body, 0)


SCAN_UNROLL = 8


def _scan_hist(g_ref, nb, target):
    lanes = _iota()
    step = L * SCAN_UNROLL

    def body(c, carry):
        found, chunk_, below, running = carry
        vs = [g_ref[pl.ds(c * step + u * L, L)] for u in range(SCAN_UNROLL)]
        sums = [jnp.broadcast_to(jnp.sum(v), (L,)) for v in vs]
        for u in range(SCAN_UNROLL):
            after = running + sums[u]
            crossed = after >= target
            is_first = jnp.logical_and(crossed, jnp.logical_not(found))
            chunk_ = jnp.where(is_first, c * SCAN_UNROLL + u, chunk_)
            below = jnp.where(is_first, running, below)
            found = jnp.logical_or(found, crossed)
            running = after
        return found, chunk_, below, running

    z = jnp.zeros((L,), jnp.int32)
    fz = jnp.zeros((L,), jnp.bool_)
    _, chunk_, below, _ = lax.fori_loop(
        0, nb // step, body, (fz, z, z, z)
    )
    cc = jnp.sum(jnp.where(lanes == 0, chunk_, 0))
    v = g_ref[pl.ds(cc * L, L)]
    cs = plsc.cumsum(v)
    crossed = (below + cs) >= target
    cnt = plsc.all_reduce_population_count(crossed)
    lane = L - cnt
    below_bin = below + jnp.broadcast_to(
        jnp.sum(jnp.where(lanes == lane, cs - v, 0)), (L,)
    )
    bin_ = chunk_ * L + lane
    return bin_, target - below_bin


def _load_slice(hbm_ref, vmem_ref, base, sem):
    return pltpu.async_copy(hbm_ref.at[pl.ds(base, E)], vmem_ref, sem)


def _worker_ids():
    cid = lax.axis_index("c")
    sid = lax.axis_index("s")
    wid = cid * NS + sid
    return cid, sid, wid


@functools.partial(
    pl.kernel,
    out_type=jax.ShapeDtypeStruct((NC, NB1), jnp.int32),
    mesh=_mesh,
    compiler_params=pltpu.CompilerParams(needs_layout_passes=False),
    scratch_types=[
        pltpu.VMEM((E,), jnp.float32),
        pltpu.VMEM((L * (NB1 + 1),), jnp.int32),
        pltpu.VMEM((NS, NB1), jnp.int32),
        pltpu.VMEM((NB1,), jnp.int32),
        pltpu.VMEM_SHARED((NS, NB1), jnp.int32),
        pltpu.SemaphoreType.DMA,
    ],
)
def _k1(loss_hbm, out_hbm, data_ref, hist_ref, stage_ref, red_ref, shared_ref, sem):
    cid, sid, wid = _worker_ids()
    dma = _load_slice(loss_hbm, data_ref, wid * E, sem)
    _zero_hist(hist_ref, NB1)
    dma.wait()
    _histogram(data_ref, hist_ref, NB1, 21)
    _stride_reduce(hist_ref, red_ref, NB1)
    _publish_hist(sid, cid, red_ref, shared_ref, stage_ref, out_hbm, NB1, sem)


@functools.partial(
    pl.kernel,
    out_type=(
        jax.ShapeDtypeStruct((NC, NB2), jnp.int32),
        jax.ShapeDtypeStruct((L,), jnp.int32),
    ),
    mesh=_mesh,
    compiler_params=pltpu.CompilerParams(needs_layout_passes=False),
    scratch_types=[
        pltpu.VMEM((E,), jnp.float32),
        pltpu.VMEM((L * (NB2 + 1),), jnp.int32),
        pltpu.VMEM((NS, NB2), jnp.int32),
        pltpu.VMEM((NB2,), jnp.int32),
        pltpu.VMEM((NC, NB1), jnp.int32),
        pltpu.VMEM((NB1,), jnp.int32),
        pltpu.VMEM((L,), jnp.int32),
        pltpu.VMEM_SHARED((NS, NB2), jnp.int32),
        pltpu.SemaphoreType.DMA,
    ],
)
def _k2(loss_hbm, hist1_hbm, out_hbm, state_hbm,
        data_ref, hist_ref, stage_ref, red_ref, pair_ref, g_ref, st_ref, shared_ref, sem):
    cid, sid, wid = _worker_ids()
    dma = _load_slice(loss_hbm, data_ref, wid * E, sem)
    _zero_hist(hist_ref, NB2)
    _sum_cores(hist1_hbm, pair_ref, g_ref, NB1)
    b1, k1 = _scan_hist(g_ref, NB1, _splat(K_RANK))
    dma.wait()
    _histogram(data_ref, hist_ref, NB2, 10, prefix_shift=21, prefix=b1)
    _stride_reduce(hist_ref, red_ref, NB2)
    _publish_hist(sid, cid, red_ref, shared_ref, stage_ref, out_hbm, NB2, sem)

    @pl.when(jnp.logical_and(cid == 0, sid == 0))
    def _():
        st_ref[...] = jnp.where(_iota() == 0, b1, jnp.where(_iota() == 1, k1, 0))
        pltpu.sync_copy(st_ref, state_hbm)


@functools.partial(
    pl.kernel,
    out_type=(
        jax.ShapeDtypeStruct((NC, NB3), jnp.int32),
        jax.ShapeDtypeStruct((L,), jnp.int32),
    ),
    mesh=_mesh,
    compiler_params=pltpu.CompilerParams(needs_layout_passes=False),
    scratch_types=[
        pltpu.VMEM((E,), jnp.float32),
        pltpu.VMEM((L * (NB3 + 1),), jnp.int32),
        pltpu.VMEM((NS, NB3), jnp.int32),
        pltpu.VMEM((NB3,), jnp.int32),
        pltpu.VMEM((NC, NB2), jnp.int32),
        pltpu.VMEM((NB2,), jnp.int32),
        pltpu.VMEM((L,), jnp.int32),
        pltpu.VMEM_SHARED((NS, NB3), jnp.int32),
        pltpu.SemaphoreType.DMA,
    ],
)
def _k3(loss_hbm, hist2_hbm, statea_hbm, out_hbm, state_hbm,
        data_ref, hist_ref, stage_ref, red_ref, pair_ref, g_ref, st_ref, shared_ref, sem):
    cid, sid, wid = _worker_ids()
    dma = _load_slice(loss_hbm, data_ref, wid * E, sem)
    _zero_hist(hist_ref, NB3)
    pltpu.sync_copy(statea_hbm, st_ref)
    sa = st_ref[...]
    b1 = _lane_splat(sa, 0)
    k1 = _lane_splat(sa, 1)
    _sum_cores(hist2_hbm, pair_ref, g_ref, NB2)
    b2, k2 = _scan_hist(g_ref, NB2, k1)
    prefix22 = b1 * 2048 + b2
    dma.wait()
    _histogram(data_ref, hist_ref, NB3, 0, prefix_shift=10, prefix=prefix22)
    _stride_reduce(hist_ref, red_ref, NB3)
    _publish_hist(sid, cid, red_ref, shared_ref, stage_ref, out_hbm, NB3, sem)

    @pl.when(jnp.logical_and(cid == 0, sid == 0))
    def _():
        st_ref[...] = jnp.where(
            _iota() == 0, prefix22, jnp.where(_iota() == 1, k2, 0)
        )
        pltpu.sync_copy(st_ref, state_hbm)


CHUNK = 4096
NCHUNK = E // CHUNK
SEL_UNROLL = 8


@functools.partial(
    pl.kernel,
    out_type=jax.ShapeDtypeStruct((N,), jnp.float32),
    mesh=_mesh,
    compiler_params=pltpu.CompilerParams(needs_layout_passes=False),
    scratch_types=[
        pltpu.VMEM((2, CHUNK), jnp.float32),
        pltpu.VMEM((2, CHUNK), jnp.float32),
        pltpu.VMEM((2, CHUNK), jnp.float32),
        pltpu.VMEM((2, CHUNK), jnp.float32),
        pltpu.VMEM((2, CHUNK), jnp.float32),
        pltpu.VMEM((2, CHUNK), jnp.float32),
        pltpu.VMEM((NC, NB3), jnp.int32),
        pltpu.VMEM((NB3,), jnp.int32),
        pltpu.VMEM((L,), jnp.int32),
        pltpu.SemaphoreType.DMA,
        pltpu.SemaphoreType.DMA,
        pltpu.SemaphoreType.DMA,
        pltpu.SemaphoreType.DMA,
    ],
)
def _k4(ol_hbm, nl_hbm, ov_hbm, cp_hbm, ru_hbm, hist3_hbm, stateb_hbm, out_hbm,
        bol_ref, bnl_ref, bov_ref, bcp_ref, bru_ref, bout_ref,
        pair_ref, g_ref, st_ref, semA, semB, osemA, osemB):
    cid, sid, wid = _worker_ids()
    srcs = (ol_hbm, nl_hbm, ov_hbm, cp_hbm, ru_hbm)
    sems = (semA, semB)
    osems = (osemA, osemB)

    bufs = (bol_ref, bnl_ref, bov_ref, bcp_ref, bru_ref)

    def issue(ch):
        p = ch % 2
        base = wid * E + ch * CHUNK
        return [
            pltpu.async_copy(src.at[pl.ds(base, CHUNK)], buf.at[p], sems[p])
            for src, buf in zip(srcs, bufs)
        ]

    handles = issue(0)
    pltpu.sync_copy(stateb_hbm, st_ref)
    sb = st_ref[...]
    prefix22 = _lane_splat(sb, 0)
    k2 = _lane_splat(sb, 1)
    _sum_cores(hist3_hbm, pair_ref, g_ref, NB3)
    b3, _ = _scan_hist(g_ref, NB3, k2)
    bits = prefix22 * 1024 + b3
    m2 = lax.shift_right_arithmetic(bits, _splat(31))
    boundary = plsc.bitcast(bits ^ (~m2 | MIN_I32), jnp.float32)
    out_handles = [None, None]
    for ch in range(NCHUNK):
        p = ch % 2
        nxt = issue(ch + 1) if ch + 1 < NCHUNK else None
        for h in handles:
            h.wait()
        if out_handles[p] is not None:
            out_handles[p].wait()

        def body(j, _):
            jb = j * (L * SEL_UNROLL)
            dss = [pl.ds(jb + u * L, L) for u in range(SEL_UNROLL)]
            ols = [bol_ref[p, ds] for ds in dss]
            nls = [bnl_ref[p, ds] for ds in dss]
            ovs = [bov_ref[p, ds] for ds in dss]
            cps = [bcp_ref[p, ds] for ds in dss]
            rus = [bru_ref[p, ds] for ds in dss]
            outs = []
            for u in range(SEL_UNROLL):
                keep = jnp.logical_and(
                    ols[u] < nls[u],
                    jnp.logical_or(
                        rus[u] >= jnp.float32(TEMP), ols[u] <= boundary
                    ),
                )
                outs.append(jnp.where(keep, ovs[u], cps[u]))
            for u, ds in enumerate(dss):
                bout_ref[p, ds] = outs[u]
            return 0

        lax.fori_loop(0, CHUNK // (L * SEL_UNROLL), body, 0)
        base = wid * E + ch * CHUNK
        out_handles[p] = pltpu.async_copy(
            bout_ref.at[p], out_hbm.at[pl.ds(base, CHUNK)], osems[p]
        )
        handles = nxt
    for oh in out_handles:
        if oh is not None:
            oh.wait()


def kernel(old_loss, new_loss, old_values, current_param, rand_u):
    hist1 = _k1(old_loss)
    hist2, state_a = _k2(old_loss, hist1)
    hist3, state_b = _k3(old_loss, hist2, state_a)
    return _k4(old_loss, new_loss, old_values, current_param, rand_u,
               hist3, state_b)


# speedup vs baseline: 1.0428x; 1.0428x over previous
import functools

import jax
import jax.numpy as jnp
import numpy as np
from jax import lax
from jax.experimental import pallas as pl
from jax.experimental.pallas import tpu as pltpu, tpu_sc as plsc

N = 1048576
K_RANK = int(N * 0.3)
TEMP = 0.05
NC = 2
NS = 16
NW = NC * NS
E = N // NW
L = 16
NB1 = 2048
NB2 = 2048
NB3 = 1024
MIN_I32 = np.int32(-2147483648)

_mesh = plsc.VectorSubcoreMesh(core_axis_name="c", subcore_axis_name="s")


def _iota():
    return lax.broadcasted_iota(jnp.int32, (L,), 0)


def _splat(x, dtype=jnp.int32):
    return jnp.broadcast_to(jnp.asarray(x, dtype), (L,))


def _lane_splat(v, i):
    return jnp.broadcast_to(jnp.sum(jnp.where(_iota() == i, v, 0)), (L,))


def _sortable(x):
    u = plsc.bitcast(x, jnp.int32)
    m = lax.shift_right_arithmetic(u, _splat(31))
    return u ^ (m | MIN_I32)


def _srl(x, n):
    return lax.shift_right_logical(x, _splat(n))


def _hstride(nb):
    return nb + 1


def _zero_hist(hist_ref, nb):
    zeros = jnp.zeros((L,), jnp.int32)
    nwords = L * _hstride(nb)
    nfull = nwords // (L * L)

    def body(c, _):
        for u in range(L):
            hist_ref[pl.ds(c * (L * L) + u * L, L)] = zeros
        return 0

    lax.fori_loop(0, nfull, body, 0)
    for r in range(nfull * L * L, nwords, L):
        hist_ref[pl.ds(r, L)] = zeros


HIST_UNROLL = 32


def _histogram(data_ref, hist_ref, nb, shift, prefix_shift=None, prefix=None):
    lane_base = _iota() * _hstride(nb)
    ones = jnp.ones((L,), jnp.int32)
    step = L * HIST_UNROLL

    def body(i, _):
        base = i * step
        xs = [data_ref[pl.ds(base + u * L, L)] for u in range(HIST_UNROLL)]
        ss = [_sortable(x) for x in xs]
        bs = [lane_base + (_srl(s, shift) & (nb - 1)) for s in ss]
        if prefix is None:
            for b in bs:
                plsc.addupdate_scatter(hist_ref, [b], ones)
        else:
            ms = [_srl(s, prefix_shift) == prefix for s in ss]
            for b, m in zip(bs, ms):
                plsc.addupdate_scatter(hist_ref, [b], ones, mask=m)
        return 0

    lax.fori_loop(0, E // step, body, 0)


def _stride_reduce(hist_ref, red_ref, nb):
    hs = _hstride(nb)

    def body(c, _):
        acc = hist_ref[pl.ds(c * L, L)]
        for l in range(1, L):
            acc = acc + hist_ref[pl.ds(c * L + l * hs, L)]
        red_ref[pl.ds(c * L, L)] = acc
        return 0

    lax.fori_loop(0, nb // L, body, 0)


def _grid_reduce(grid_ref, red_ref, nb):
    def body(c, _):
        acc = grid_ref[0, pl.ds(c * L, L)]
        for l in range(1, L):
            acc = acc + grid_ref[l, pl.ds(c * L, L)]
        red_ref[pl.ds(c * L, L)] = acc
        return 0

    lax.fori_loop(0, nb // L, body, 0)


def _publish_hist(sid, cid, red_ref, shared_ref, stage_ref, out_hbm, nb,
                  sem):
    sl = nb // NS
    pltpu.sync_copy(red_ref, shared_ref.at[sid])
    plsc.subcore_barrier()
    hs = [
        pltpu.async_copy(
            shared_ref.at[l, pl.ds(sid * sl, sl)],
            stage_ref.at[l, pl.ds(0, sl)],
            sem,
        )
        for l in range(NS)
    ]
    for h in hs:
        h.wait()

    def body(c, _):
        acc = stage_ref[0, pl.ds(c * L, L)]
        for l in range(1, NS):
            acc = acc + stage_ref[l, pl.ds(c * L, L)]
        red_ref[pl.ds(c * L, L)] = acc
        return 0

    lax.fori_loop(0, sl // L, body, 0)
    pltpu.sync_copy(
        red_ref.at[pl.ds(0, sl)], out_hbm.at[cid, pl.ds(sid * sl, sl)]
    )


SUM_UNROLL = 8


def _sum_cores(hist_hbm, pair_ref, g_ref, nb):
    pltpu.sync_copy(hist_hbm, pair_ref)
    step = L * SUM_UNROLL

    def body(c, _):
        dss = [pl.ds(c * step + u * L, L) for u in range(SUM_UNROLL)]
        a = [pair_ref[0, ds] for ds in dss]
        b = [pair_ref[1, ds] for ds in dss]
        for u, ds in enumerate(dss):
            g_ref[ds] = a[u] + b[u]
        return 0

    lax.fori_loop(0, nb // step, body, 0)


SCAN_UNROLL = 8


def _scan_hist(g_ref, nb, target):
    lanes = _iota()
    step = L * SCAN_UNROLL

    def body(c, carry):
        found, chunk_, below, running = carry
        vs = [g_ref[pl.ds(c * step + u * L, L)] for u in range(SCAN_UNROLL)]
        sums = [jnp.broadcast_to(jnp.sum(v), (L,)) for v in vs]
        for u in range(SCAN_UNROLL):
            after = running + sums[u]
            crossed = after >= target
            is_first = jnp.logical_and(crossed, jnp.logical_not(found))
            chunk_ = jnp.where(is_first, c * SCAN_UNROLL + u, chunk_)
            below = jnp.where(is_first, running, below)
            found = jnp.logical_or(found, crossed)
            running = after
        return found, chunk_, below, running

    z = jnp.zeros((L,), jnp.int32)
    fz = jnp.zeros((L,), jnp.bool_)
    _, chunk_, below, _ = lax.fori_loop(
        0, nb // step, body, (fz, z, z, z)
    )
    cc = jnp.sum(jnp.where(lanes == 0, chunk_, 0))
    v = g_ref[pl.ds(cc * L, L)]
    cs = plsc.cumsum(v)
    crossed = (below + cs) >= target
    cnt = plsc.all_reduce_population_count(crossed)
    lane = L - cnt
    below_bin = below + jnp.broadcast_to(
        jnp.sum(jnp.where(lanes == lane, cs - v, 0)), (L,)
    )
    bin_ = chunk_ * L + lane
    return bin_, target - below_bin


def _load_slice(hbm_ref, vmem_ref, base, sem):
    return pltpu.async_copy(hbm_ref.at[pl.ds(base, E)], vmem_ref, sem)


def _worker_ids():
    cid = lax.axis_index("c")
    sid = lax.axis_index("s")
    wid = cid * NS + sid
    return cid, sid, wid


@functools.partial(
    pl.kernel,
    out_type=jax.ShapeDtypeStruct((NC, NB1), jnp.int32),
    mesh=_mesh,
    compiler_params=pltpu.CompilerParams(needs_layout_passes=False),
    scratch_types=[
        pltpu.VMEM((E,), jnp.float32),
        pltpu.VMEM((L * (NB1 + 1),), jnp.int32),
        pltpu.VMEM((NS, NB1), jnp.int32),
        pltpu.VMEM((NB1,), jnp.int32),
        pltpu.VMEM_SHARED((NS, NB1), jnp.int32),
        pltpu.SemaphoreType.DMA,
    ],
)
def _k1(loss_hbm, out_hbm, data_ref, hist_ref, stage_ref, red_ref, shared_ref, sem):
    cid, sid, wid = _worker_ids()
    dma = _load_slice(loss_hbm, data_ref, wid * E, sem)
    _zero_hist(hist_ref, NB1)
    dma.wait()
    _histogram(data_ref, hist_ref, NB1, 21)
    _stride_reduce(hist_ref, red_ref, NB1)
    _publish_hist(sid, cid, red_ref, shared_ref, stage_ref, out_hbm, NB1, sem)


@functools.partial(
    pl.kernel,
    out_type=(
        jax.ShapeDtypeStruct((NC, NB2), jnp.int32),
        jax.ShapeDtypeStruct((L,), jnp.int32),
    ),
    mesh=_mesh,
    compiler_params=pltpu.CompilerParams(needs_layout_passes=False),
    scratch_types=[
        pltpu.VMEM((E,), jnp.float32),
        pltpu.VMEM((L * (NB2 + 1),), jnp.int32),
        pltpu.VMEM((NS, NB2), jnp.int32),
        pltpu.VMEM((NB2,), jnp.int32),
        pltpu.VMEM((NC, NB1), jnp.int32),
        pltpu.VMEM((NB1,), jnp.int32),
        pltpu.VMEM((L,), jnp.int32),
        pltpu.VMEM_SHARED((NS, NB2), jnp.int32),
        pltpu.SemaphoreType.DMA,
    ],
)
def _k2(loss_hbm, hist1_hbm, out_hbm, state_hbm,
        data_ref, hist_ref, stage_ref, red_ref, pair_ref, g_ref, st_ref, shared_ref, sem):
    cid, sid, wid = _worker_ids()
    dma = _load_slice(loss_hbm, data_ref, wid * E, sem)
    _zero_hist(hist_ref, NB2)
    _sum_cores(hist1_hbm, pair_ref, g_ref, NB1)
    b1, k1 = _scan_hist(g_ref, NB1, _splat(K_RANK))
    dma.wait()
    _histogram(data_ref, hist_ref, NB2, 10, prefix_shift=21, prefix=b1)
    _stride_reduce(hist_ref, red_ref, NB2)
    _publish_hist(sid, cid, red_ref, shared_ref, stage_ref, out_hbm, NB2, sem)

    @pl.when(jnp.logical_and(cid == 0, sid == 0))
    def _():
        st_ref[...] = jnp.where(_iota() == 0, b1, jnp.where(_iota() == 1, k1, 0))
        pltpu.sync_copy(st_ref, state_hbm)


@functools.partial(
    pl.kernel,
    out_type=(
        jax.ShapeDtypeStruct((NC, NB3), jnp.int32),
        jax.ShapeDtypeStruct((L,), jnp.int32),
    ),
    mesh=_mesh,
    compiler_params=pltpu.CompilerParams(needs_layout_passes=False),
    scratch_types=[
        pltpu.VMEM((E,), jnp.float32),
        pltpu.VMEM((L * (NB3 + 1),), jnp.int32),
        pltpu.VMEM((NS, NB3), jnp.int32),
        pltpu.VMEM((NB3,), jnp.int32),
        pltpu.VMEM((NC, NB2), jnp.int32),
        pltpu.VMEM((NB2,), jnp.int32),
        pltpu.VMEM((L,), jnp.int32),
        pltpu.VMEM_SHARED((NS, NB3), jnp.int32),
        pltpu.SemaphoreType.DMA,
    ],
)
def _k3(loss_hbm, hist2_hbm, statea_hbm, out_hbm, state_hbm,
        data_ref, hist_ref, stage_ref, red_ref, pair_ref, g_ref, st_ref, shared_ref, sem):
    cid, sid, wid = _worker_ids()
    dma = _load_slice(loss_hbm, data_ref, wid * E, sem)
    _zero_hist(hist_ref, NB3)
    pltpu.sync_copy(statea_hbm, st_ref)
    sa = st_ref[...]
    b1 = _lane_splat(sa, 0)
    k1 = _lane_splat(sa, 1)
    _sum_cores(hist2_hbm, pair_ref, g_ref, NB2)
    b2, k2 = _scan_hist(g_ref, NB2, k1)
    prefix22 = b1 * 2048 + b2
    dma.wait()
    _histogram(data_ref, hist_ref, NB3, 0, prefix_shift=10, prefix=prefix22)
    _stride_reduce(hist_ref, red_ref, NB3)
    _publish_hist(sid, cid, red_ref, shared_ref, stage_ref, out_hbm, NB3, sem)

    @pl.when(jnp.logical_and(cid == 0, sid == 0))
    def _():
        st_ref[...] = jnp.where(
            _iota() == 0, prefix22, jnp.where(_iota() == 1, k2, 0)
        )
        pltpu.sync_copy(st_ref, state_hbm)


CHUNK = 8192
NCHUNK = E // CHUNK
SEL_UNROLL = 8


@functools.partial(
    pl.kernel,
    out_type=jax.ShapeDtypeStruct((N,), jnp.float32),
    mesh=_mesh,
    compiler_params=pltpu.CompilerParams(needs_layout_passes=False),
    scratch_types=[
        pltpu.VMEM((2, CHUNK), jnp.float32),
        pltpu.VMEM((2, CHUNK), jnp.float32),
        pltpu.VMEM((2, CHUNK), jnp.float32),
        pltpu.VMEM((2, CHUNK), jnp.float32),
        pltpu.VMEM((2, CHUNK), jnp.float32),
        pltpu.VMEM((2, CHUNK), jnp.float32),
        pltpu.VMEM((NC, NB3), jnp.int32),
        pltpu.VMEM((NB3,), jnp.int32),
        pltpu.VMEM((L,), jnp.int32),
        pltpu.SemaphoreType.DMA,
        pltpu.SemaphoreType.DMA,
        pltpu.SemaphoreType.DMA,
        pltpu.SemaphoreType.DMA,
    ],
)
def _k4(ol_hbm, nl_hbm, ov_hbm, cp_hbm, ru_hbm, hist3_hbm, stateb_hbm, out_hbm,
        bol_ref, bnl_ref, bov_ref, bcp_ref, bru_ref, bout_ref,
        pair_ref, g_ref, st_ref, semA, semB, osemA, osemB):
    cid, sid, wid = _worker_ids()
    srcs = (ol_hbm, nl_hbm, ov_hbm, cp_hbm, ru_hbm)
    sems = (semA, semB)
    osems = (osemA, osemB)

    bufs = (bol_ref, bnl_ref, bov_ref, bcp_ref, bru_ref)

    def issue(ch):
        p = ch % 2
        base = wid * E + ch * CHUNK
        return [
            pltpu.async_copy(src.at[pl.ds(base, CHUNK)], buf.at[p], sems[p])
            for src, buf in zip(srcs, bufs)
        ]

    handles = issue(0)
    pltpu.sync_copy(stateb_hbm, st_ref)
    sb = st_ref[...]
    prefix22 = _lane_splat(sb, 0)
    k2 = _lane_splat(sb, 1)
    _sum_cores(hist3_hbm, pair_ref, g_ref, NB3)
    b3, _ = _scan_hist(g_ref, NB3, k2)
    bits = prefix22 * 1024 + b3
    m2 = lax.shift_right_arithmetic(bits, _splat(31))
    boundary = plsc.bitcast(bits ^ (~m2 | MIN_I32), jnp.float32)
    out_handles = [None, None]
    for ch in range(NCHUNK):
        p = ch % 2
        nxt = issue(ch + 1) if ch + 1 < NCHUNK else None
        for h in handles:
            h.wait()
        if out_handles[p] is not None:
            out_handles[p].wait()

        def body(j, _):
            jb = j * (L * SEL_UNROLL)
            dss = [pl.ds(jb + u * L, L) for u in range(SEL_UNROLL)]
            ols = [bol_ref[p, ds] for ds in dss]
            nls = [bnl_ref[p, ds] for ds in dss]
            ovs = [bov_ref[p, ds] for ds in dss]
            cps = [bcp_ref[p, ds] for ds in dss]
            rus = [bru_ref[p, ds] for ds in dss]
            outs = []
            for u in range(SEL_UNROLL):
                keep = jnp.logical_and(
                    ols[u] < nls[u],
                    jnp.logical_or(
                        rus[u] >= jnp.float32(TEMP), ols[u] <= boundary
                    ),
                )
                outs.append(jnp.where(keep, ovs[u], cps[u]))
            for u, ds in enumerate(dss):
                bout_ref[p, ds] = outs[u]
            return 0

        lax.fori_loop(0, CHUNK // (L * SEL_UNROLL), body, 0)
        base = wid * E + ch * CHUNK
        out_handles[p] = pltpu.async_copy(
            bout_ref.at[p], out_hbm.at[pl.ds(base, CHUNK)], osems[p]
        )
        handles = nxt
    for oh in out_handles:
        if oh is not None:
            oh.wait()


def kernel(old_loss, new_loss, old_values, current_param, rand_u):
    hist1 = _k1(old_loss)
    hist2, state_a = _k2(old_loss, hist1)
    hist3, state_b = _k3(old_loss, hist2, state_a)
    return _k4(old_loss, new_loss, old_values, current_param, rand_u,
               hist3, state_b)
